# baseline (device time: 35343 ns/iter reference)
import jax
import jax.numpy as jnp
from jax import lax
from jax.experimental import pallas as pl
from jax.experimental.pallas import tpu as pltpu

C = 8


def kernel(x, W):
    t, d = x.shape
    _, v_loc = W.shape
    v_tot = 2 * v_loc
    cs = v_loc // C

    def body(x_ref, w_ref, out_ref, send_buf, recv_buf, s_send, s_recv,
             send_sems, recv_sems, s_send_sem, s_recv_sem):
        my_x = lax.axis_index("x")
        my_y = lax.axis_index("y")
        my_z = lax.axis_index("z")
        partner = (1 - my_x, my_y, my_z)

        barrier_sem = pltpu.get_barrier_semaphore()
        pl.semaphore_signal(
            barrier_sem, inc=1,
            device_id=partner, device_id_type=pl.DeviceIdType.MESH,
        )
        pl.semaphore_wait(barrier_sem, 1)

        x_bf = x_ref[:, :].astype(jnp.bfloat16)

        rdmas = []
        s_loc = jnp.zeros((t, 1), jnp.float32)
        for k in range(C):
            lk = jnp.dot(
                x_bf,
                w_ref[:, k * cs:(k + 1) * cs].astype(jnp.bfloat16),
                preferred_element_type=jnp.float32,
            )
            ek = jnp.exp(lk)
            s_loc = s_loc + jnp.sum(ek, axis=1, keepdims=True)
            send_buf[k] = ek.astype(jnp.bfloat16)
            rdma = pltpu.make_async_remote_copy(
                src_ref=send_buf.at[k],
                dst_ref=recv_buf.at[k],
                send_sem=send_sems.at[k],
                recv_sem=recv_sems.at[k],
                device_id=partner,
                device_id_type=pl.DeviceIdType.MESH,
            )
            rdma.start()
            rdmas.append(rdma)

        s_send[:, :] = s_loc
        s_rdma = pltpu.make_async_remote_copy(
            src_ref=s_send,
            dst_ref=s_recv,
            send_sem=s_send_sem,
            recv_sem=s_recv_sem,
            device_id=partner,
            device_id_type=pl.DeviceIdType.MESH,
        )
        s_rdma.start()
        s_rdma.wait_recv()

        recip = 1.0 / (s_loc + s_recv[:, :])
        my_off = my_x * v_loc
        other_off = (1 - my_x) * v_loc
        for k in range(C):
            out_ref[:, pl.ds(my_off + k * cs, cs)] = (
                send_buf[k].astype(jnp.float32) * recip
            ).astype(jnp.bfloat16)
        for k in range(C):
            rdmas[k].wait_recv()
            out_ref[:, pl.ds(other_off + k * cs, cs)] = (
                recv_buf[k].astype(jnp.float32) * recip
            ).astype(jnp.bfloat16)

        for k in range(C):
            rdmas[k].wait_send()
        s_rdma.wait_send()

    return pl.pallas_call(
        body,
        out_shape=jax.ShapeDtypeStruct((t, v_tot), jnp.bfloat16),
        in_specs=[
            pl.BlockSpec(memory_space=pltpu.VMEM),
            pl.BlockSpec(memory_space=pltpu.VMEM),
        ],
        out_specs=pl.BlockSpec(memory_space=pltpu.VMEM),
        scratch_shapes=[
            pltpu.VMEM((C, t, cs), jnp.bfloat16),
            pltpu.VMEM((C, t, cs), jnp.bfloat16),
            pltpu.VMEM((t, 1), jnp.float32),
            pltpu.VMEM((t, 1), jnp.float32),
            pltpu.SemaphoreType.DMA((C,)),
            pltpu.SemaphoreType.DMA((C,)),
            pltpu.SemaphoreType.DMA,
            pltpu.SemaphoreType.DMA,
        ],
        compiler_params=pltpu.CompilerParams(collective_id=0),
    )(x, W)


# device time: 34619 ns/iter; 1.0209x vs baseline; 1.0209x over previous
import jax
import jax.numpy as jnp
from jax import lax
from jax.experimental import pallas as pl
from jax.experimental.pallas import tpu as pltpu

C = 8


def kernel(x, W):
    t, d = x.shape
    _, v_loc = W.shape
    v_tot = 2 * v_loc
    cs = v_loc // C

    def body(x_ref, w_ref, out_ref, send_buf, recv_buf, s_send, s_recv,
             send_sems, recv_sems, s_send_sem, s_recv_sem):
        my_x = lax.axis_index("x")
        my_y = lax.axis_index("y")
        my_z = lax.axis_index("z")
        partner = (1 - my_x, my_y, my_z)

        barrier_sem = pltpu.get_barrier_semaphore()
        pl.semaphore_signal(
            barrier_sem, inc=1,
            device_id=partner, device_id_type=pl.DeviceIdType.MESH,
        )
        pl.semaphore_wait(barrier_sem, 1)

        x_bf = x_ref[:, :].astype(jnp.bfloat16)

        rdmas = []
        s_loc = jnp.zeros((t, 1), jnp.float32)
        for k in range(C):
            lk = jnp.dot(
                x_bf,
                w_ref[:, k * cs:(k + 1) * cs].astype(jnp.bfloat16),
                preferred_element_type=jnp.float32,
            )
            ek = jnp.exp(lk)
            s_loc = s_loc + jnp.sum(ek, axis=1, keepdims=True)
            send_buf[k] = ek.astype(jnp.bfloat16)
            rdma = pltpu.make_async_remote_copy(
                src_ref=send_buf.at[k],
                dst_ref=recv_buf.at[k],
                send_sem=send_sems.at[k],
                recv_sem=recv_sems.at[k],
                device_id=partner,
                device_id_type=pl.DeviceIdType.MESH,
            )
            rdmas.append(rdma)
            if k < C - 2:
                rdma.start()

        s_send[:, :] = s_loc
        s_rdma = pltpu.make_async_remote_copy(
            src_ref=s_send,
            dst_ref=s_recv,
            send_sem=s_send_sem,
            recv_sem=s_recv_sem,
            device_id=partner,
            device_id_type=pl.DeviceIdType.MESH,
        )
        s_rdma.start()
        rdmas[C - 2].start()
        rdmas[C - 1].start()

        s_rdma.wait_recv()

        recip = 1.0 / (s_loc + s_recv[:, :])
        my_off = my_x * v_loc
        other_off = (1 - my_x) * v_loc
        for k in range(C):
            out_ref[:, pl.ds(my_off + k * cs, cs)] = (
                send_buf[k].astype(jnp.float32) * recip
            ).astype(jnp.bfloat16)
        for k in range(C):
            rdmas[k].wait_recv()
            out_ref[:, pl.ds(other_off + k * cs, cs)] = (
                recv_buf[k].astype(jnp.float32) * recip
            ).astype(jnp.bfloat16)

        for k in range(C):
            rdmas[k].wait_send()
        s_rdma.wait_send()

    return pl.pallas_call(
        body,
        out_shape=jax.ShapeDtypeStruct((t, v_tot), jnp.bfloat16),
        in_specs=[
            pl.BlockSpec(memory_space=pltpu.VMEM),
            pl.BlockSpec(memory_space=pltpu.VMEM),
        ],
        out_specs=pl.BlockSpec(memory_space=pltpu.VMEM),
        scratch_shapes=[
            pltpu.VMEM((C, t, cs), jnp.bfloat16),
            pltpu.VMEM((C, t, cs), jnp.bfloat16),
            pltpu.VMEM((t, 1), jnp.float32),
            pltpu.VMEM((t, 1), jnp.float32),
            pltpu.SemaphoreType.DMA((C,)),
            pltpu.SemaphoreType.DMA((C,)),
            pltpu.SemaphoreType.DMA,
            pltpu.SemaphoreType.DMA,
        ],
        compiler_params=pltpu.CompilerParams(collective_id=0),
    )(x, W)


# device time: 33903 ns/iter; 1.0425x vs baseline; 1.0211x over previous
import jax
import jax.numpy as jnp
from jax import lax
from jax.experimental import pallas as pl
from jax.experimental.pallas import tpu as pltpu

C = 8


def kernel(x, W):
    t, d = x.shape
    _, v_loc = W.shape
    v_tot = 2 * v_loc
    cs = v_loc // C

    def body(x_ref, w_ref, out_ref, send_buf, recv_buf, s_send, s_recv,
             send_sems, recv_sems, s_send_sem, s_recv_sem):
        my_x = lax.axis_index("x")
        my_y = lax.axis_index("y")
        my_z = lax.axis_index("z")
        partner = (1 - my_x, my_y, my_z)

        barrier_sem = pltpu.get_barrier_semaphore()
        pl.semaphore_signal(
            barrier_sem, inc=1,
            device_id=partner, device_id_type=pl.DeviceIdType.MESH,
        )
        pl.semaphore_wait(barrier_sem, 1)

        x_bf = x_ref[:, :].astype(jnp.bfloat16)

        rdmas = []
        s_loc = jnp.zeros((t, 1), jnp.float32)
        for k in range(C):
            lk = jnp.dot(
                x_bf,
                w_ref[:, k * cs:(k + 1) * cs].astype(jnp.bfloat16),
                preferred_element_type=jnp.float32,
            )
            ek = jnp.exp(lk)
            s_loc = s_loc + jnp.sum(ek, axis=1, keepdims=True)
            send_buf[k] = ek.astype(jnp.bfloat16)
            rdma = pltpu.make_async_remote_copy(
                src_ref=send_buf.at[k],
                dst_ref=recv_buf.at[k],
                send_sem=send_sems.at[k],
                recv_sem=recv_sems.at[k],
                device_id=partner,
                device_id_type=pl.DeviceIdType.MESH,
            )
            rdmas.append(rdma)
            if k < C - 2:
                rdma.start()

        s_send[:, :] = s_loc.astype(jnp.bfloat16)
        s_rdma = pltpu.make_async_remote_copy(
            src_ref=s_send,
            dst_ref=s_recv,
            send_sem=s_send_sem,
            recv_sem=s_recv_sem,
            device_id=partner,
            device_id_type=pl.DeviceIdType.MESH,
        )
        s_rdma.start()
        rdmas[C - 2].start()
        rdmas[C - 1].start()

        s_rdma.wait_recv()

        recip = 1.0 / (s_loc + s_recv[:, :].astype(jnp.float32))
        my_off = my_x * v_loc
        other_off = (1 - my_x) * v_loc
        for k in range(C):
            out_ref[:, pl.ds(my_off + k * cs, cs)] = (
                send_buf[k].astype(jnp.float32) * recip
            ).astype(jnp.bfloat16)
        for k in range(C):
            rdmas[k].wait_recv()
            out_ref[:, pl.ds(other_off + k * cs, cs)] = (
                recv_buf[k].astype(jnp.float32) * recip
            ).astype(jnp.bfloat16)

        for k in range(C):
            rdmas[k].wait_send()
        s_rdma.wait_send()

    return pl.pallas_call(
        body,
        out_shape=jax.ShapeDtypeStruct((t, v_tot), jnp.bfloat16),
        in_specs=[
            pl.BlockSpec(memory_space=pltpu.VMEM),
            pl.BlockSpec(memory_space=pltpu.VMEM),
        ],
        out_specs=pl.BlockSpec(memory_space=pltpu.VMEM),
        scratch_shapes=[
            pltpu.VMEM((C, t, cs), jnp.bfloat16),
            pltpu.VMEM((C, t, cs), jnp.bfloat16),
            pltpu.VMEM((t, 1), jnp.bfloat16),
            pltpu.VMEM((t, 1), jnp.bfloat16),
            pltpu.SemaphoreType.DMA((C,)),
            pltpu.SemaphoreType.DMA((C,)),
            pltpu.SemaphoreType.DMA,
            pltpu.SemaphoreType.DMA,
        ],
        compiler_params=pltpu.CompilerParams(collective_id=0),
    )(x, W)
